# Initial kernel scaffold; baseline (speedup 1.0000x reference)
import jax
import jax.numpy as jnp
from jax.experimental import pallas as pl


def kernel(inputs, position_embeddings):
    raise NotImplementedError("write your pallas kernel here")


# speedup vs baseline: 1.4896x; 1.4896x over previous
import functools

import jax
import jax.numpy as jnp
from jax import lax
from jax.experimental import pallas as pl
from jax.experimental.pallas import tpu as pltpu
from jax.experimental.pallas import tpu_sc as plsc

ALPHA = 0.4
SEQ = 2048
FEAT = 768
BATCH = 4
CHUNKS = 4
NC, NS = 2, 16
NW = NC * NS
ROWS_W = SEQ // NW
HALF = ROWS_W // 2
NT = FEAT // 16
OUT_ROWS = BATCH * CHUNKS * SEQ


def _build_sc_call():
    mesh = plsc.VectorSubcoreMesh(core_axis_name="c", subcore_axis_name="s")

    @functools.partial(
        pl.kernel,
        mesh=mesh,
        out_type=jax.ShapeDtypeStruct((OUT_ROWS, FEAT), jnp.float32),
        scratch_types=[
            pltpu.VMEM((HALF, FEAT), jnp.float32),
            pltpu.VMEM((4, FEAT), jnp.float32),
            pltpu.VMEM((4, FEAT), jnp.float32),
            pltpu.VMEM((HALF, FEAT), jnp.float32),
            pltpu.VMEM((HALF, FEAT), jnp.float32),
            pltpu.VMEM((HALF, FEAT), jnp.float32),
        ],
    )
    def pe_kernel(table, out, seg, head, cq, b1, b2, b3):
        w = lax.axis_index("s") * NC + lax.axis_index("c")
        pltpu.sync_copy(table.at[pl.ds(0, 4), :], head)

        a = ALPHA
        s = a / (1.0 - a)
        for q in range(1, CHUNKS):
            for t in range(NT):
                sl = pl.ds(t * 16, 16)
                p0 = head[0, sl]
                pq = head[q, sl]
                cq[q, sl] = s * (pq - a * p0) - a * p0

        bufs = {1: b1, 2: b2, 3: b3}
        for h in range(2):
            j0 = w * ROWS_W + h * HALF
            pltpu.sync_copy(table.at[pl.ds(j0, HALF), :], seg)
            for b in range(BATCH):
                pltpu.sync_copy(seg, out.at[pl.ds(b * CHUNKS * SEQ + j0, HALF), :])
            for q in range(1, CHUNKS):
                buf = bufs[q]

                def body(r, carry, _buf=buf, _q=q):
                    for t in range(NT):
                        sl = pl.ds(t * 16, 16)
                        _buf[r, sl] = seg[r, sl] + cq[_q, sl]
                    return carry

                lax.fori_loop(0, HALF, body, None)
                for b in range(BATCH):
                    dst = out.at[pl.ds(b * CHUNKS * SEQ + q * SEQ + j0, HALF), :]
                    pltpu.sync_copy(buf, dst)

    return pe_kernel


_sc_call = jax.jit(_build_sc_call())


def kernel(inputs, position_embeddings):
    pe = _sc_call(position_embeddings)
    return pe.reshape(inputs.shape)


# speedup vs baseline: 1.9637x; 1.3183x over previous
import functools

import jax
import jax.numpy as jnp
from jax import lax
from jax.experimental import pallas as pl
from jax.experimental.pallas import tpu as pltpu
from jax.experimental.pallas import tpu_sc as plsc

ALPHA = 0.4
SEQ = 2048
FEAT = 768
BATCH = 4
CHUNKS = 4
NC, NS = 2, 16
NW = NC * NS
ROWS_W = SEQ // NW
HALF = ROWS_W // 2
NT = FEAT // 16
OUT_ROWS = BATCH * CHUNKS * SEQ


def _build_sc_call():
    mesh = plsc.VectorSubcoreMesh(core_axis_name="c", subcore_axis_name="s")

    @functools.partial(
        pl.kernel,
        mesh=mesh,
        out_type=jax.ShapeDtypeStruct((OUT_ROWS, FEAT), jnp.float32),
        scratch_types=[
            pltpu.VMEM((HALF, FEAT), jnp.float32),
            pltpu.VMEM((4, FEAT), jnp.float32),
            pltpu.VMEM((4, FEAT), jnp.float32),
            pltpu.VMEM((HALF, FEAT), jnp.float32),
            pltpu.VMEM((HALF, FEAT), jnp.float32),
            pltpu.VMEM((HALF, FEAT), jnp.float32),
            pltpu.SemaphoreType.DMA,
            pltpu.SemaphoreType.DMA,
            pltpu.SemaphoreType.DMA,
            pltpu.SemaphoreType.DMA,
        ],
    )
    def pe_kernel(table, out, seg, head, cq, b1, b2, b3, s0, s1, s2, s3):
        w = lax.axis_index("s") * NC + lax.axis_index("c")
        pltpu.sync_copy(table.at[pl.ds(0, 4), :], head)

        a = ALPHA
        s = a / (1.0 - a)
        for q in range(1, CHUNKS):
            for t in range(NT):
                sl = pl.ds(t * 16, 16)
                p0 = head[0, sl]
                pq = head[q, sl]
                cq[q, sl] = s * (pq - a * p0) - a * p0

        bufs = {1: b1, 2: b2, 3: b3}
        sems = {0: s0, 1: s1, 2: s2, 3: s3}
        pending = {}
        for h in range(2):
            j0 = w * ROWS_W + h * HALF
            if h > 0:
                for cp in pending[(h - 1, 0)]:
                    cp.wait()
            pltpu.sync_copy(table.at[pl.ds(j0, HALF), :], seg)
            pending[(h, 0)] = [
                pltpu.async_copy(
                    seg, out.at[pl.ds(b * CHUNKS * SEQ + j0, HALF), :], s0
                )
                for b in range(BATCH)
            ]
            for q in range(1, CHUNKS):
                buf = bufs[q]
                if h > 0:
                    for cp in pending[(h - 1, q)]:
                        cp.wait()

                def body(r, carry, _buf=buf, _q=q):
                    for t in range(NT):
                        sl = pl.ds(t * 16, 16)
                        _buf[r, sl] = seg[r, sl] + cq[_q, sl]
                    return carry

                lax.fori_loop(0, HALF, body, None)
                pending[(h, q)] = [
                    pltpu.async_copy(
                        buf,
                        out.at[pl.ds(b * CHUNKS * SEQ + q * SEQ + j0, HALF), :],
                        sems[q],
                    )
                    for b in range(BATCH)
                ]
        for q in range(CHUNKS):
            for cp in pending[(1, q)]:
                cp.wait()

    return pe_kernel


_sc_call = jax.jit(_build_sc_call())


def kernel(inputs, position_embeddings):
    pe = _sc_call(position_embeddings)
    return pe.reshape(inputs.shape)
